# baseline (speedup 1.0000x reference)
import functools

import jax
import jax.numpy as jnp
from jax.experimental import pallas as pl
from jax.experimental.pallas import tpu as pltpu

_R_MAX = 5.0
_P_CUT = 5.0
_AVG_NEIGH = 32.0
_NUM_BESSEL = 8
_NUM_SH = 16
_F = 32
_KF = _NUM_SH * _F

_CE = 512
_BN = 80


def _bessel_j(r):
    n = jnp.arange(1, _NUM_BESSEL + 1, dtype=jnp.float32)
    rr = jnp.clip(r, 1e-6, None)[:, None]
    return jnp.sqrt(2.0 / _R_MAX) * jnp.sin(n * jnp.pi * rr / _R_MAX) / rr


def _cutoff_f(r):
    u = r / _R_MAX
    p = _P_CUT
    f = (1.0 - (p + 1.0) * (p + 2.0) / 2.0 * u ** p
         + p * (p + 2.0) * u ** (p + 1.0)
         - p * (p + 1.0) / 2.0 * u ** (p + 2.0))
    return jnp.where(u < 1.0, f, 0.0)


def _sph_f(vec):
    r = jnp.sqrt(jnp.sum(vec * vec, axis=-1, keepdims=True) + 1e-9)
    u = vec / r
    x, y, z = u[:, 0], u[:, 1], u[:, 2]
    s3, s5, s15 = jnp.sqrt(3.0), jnp.sqrt(5.0), jnp.sqrt(15.0)
    sh = [jnp.ones_like(x),
          s3 * x, s3 * y, s3 * z,
          s15 * x * y, s15 * y * z,
          s5 / 2.0 * (3.0 * z ** 2 - 1.0),
          s15 * x * z, s15 / 2.0 * (x ** 2 - y ** 2),
          jnp.sqrt(35.0 / 8.0) * (3.0 * x ** 2 - y ** 2) * y,
          jnp.sqrt(105.0) * x * y * z,
          jnp.sqrt(21.0 / 8.0) * y * (5.0 * z ** 2 - 1.0),
          jnp.sqrt(7.0) / 2.0 * z * (5.0 * z ** 2 - 3.0),
          jnp.sqrt(21.0 / 8.0) * x * (5.0 * z ** 2 - 1.0),
          jnp.sqrt(105.0) / 2.0 * (x ** 2 - y ** 2) * z,
          jnp.sqrt(35.0 / 8.0) * (x ** 2 - 3.0 * y ** 2) * x]
    return jnp.stack(sh, axis=-1)


def _build_worklist(row_ptr, n_windows, n_blocks, ce, bn):
    w = jnp.arange(n_windows, dtype=jnp.int32)
    s_w = row_ptr[w * bn]
    e_w = row_ptr[(w + 1) * bn]
    b_start = s_w // ce
    b_end = (e_w + ce - 1) // ce
    nb = jnp.maximum(b_end - b_start, 1).astype(jnp.int32)
    off = jnp.concatenate([jnp.zeros((1,), jnp.int32),
                           jnp.cumsum(nb).astype(jnp.int32)])
    total = off[-1]
    l_max = n_blocks + n_windows
    t = jnp.arange(l_max, dtype=jnp.int32)
    wt = jnp.clip(jnp.searchsorted(off, t, side="right").astype(jnp.int32) - 1,
                  0, n_windows - 1)
    eb = jnp.clip(b_start[wt] + (t - off[wt]), 0, n_blocks - 1)
    ow = wt
    s_t = jnp.maximum(s_w[wt], eb * ce)
    e_t = jnp.minimum(e_w[wt], (eb + 1) * ce)
    valid = t < total
    s_t = jnp.where(valid, s_t, 0)
    e_t = jnp.where(valid, e_t, 0)
    ones = jnp.ones((1,), jnp.bool_)
    init_f = jnp.concatenate([ones, ow[1:] != ow[:-1]]).astype(jnp.int32)
    init_b = jnp.concatenate([ones, eb[1:] != eb[:-1]]).astype(jnp.int32)
    return eb, ow, s_t, e_t, init_f, init_b


def _dsilu(z):
    s = jax.nn.sigmoid(z)
    return s * (1.0 + z * (1.0 - s))


def _onehot(eb, ow, s, e, recv_r, bn, ce):
    recv = recv_r[0]
    base = eb * ce
    ids = base + jax.lax.broadcasted_iota(jnp.int32, (1, ce), 1)
    mask = (ids >= s) & (ids < e)
    local = recv - ow * bn
    rows = jax.lax.broadcasted_iota(jnp.int32, (bn, ce), 0)
    return ((rows == local) & mask).astype(jnp.float32)


_BF = jnp.bfloat16


def _mlp_fwd(ef, w1, b1, w2, b2, w3, b3, w4):
    z1 = jnp.dot(ef.astype(_BF), w1, preferred_element_type=jnp.float32) + b1[0:1, :]
    h1 = z1 * jax.nn.sigmoid(z1)
    z2 = jnp.dot(h1.astype(_BF), w2, preferred_element_type=jnp.float32) + b2[0:1, :]
    h2 = z2 * jax.nn.sigmoid(z2)
    z3 = jnp.dot(h2.astype(_BF), w3, preferred_element_type=jnp.float32) + b3[0:1, :]
    h3 = z3 * jax.nn.sigmoid(z3)
    w = jnp.dot(h3.astype(_BF), w4, preferred_element_type=jnp.float32)
    return z1, h1, z2, h2, z3, h3, w


def _fwd_body(eb_r, ow_r, s_r, e_r, initf_r, initb_r,
              ef_r, sh_r, upg_r, recv_r,
              w1_r, b1_r, w2_r, b2_r, w3_r, b3_r, w4_r, rm_r, tm_r,
              out_r):
    t = pl.program_id(0)
    eb = eb_r[t]
    ow = ow_r[t]
    s = s_r[t]
    e = e_r[t]
    init = initf_r[t]
    _, _, _, _, _, _, w = _mlp_fwd(ef_r[...], w1_r[...], b1_r[...], w2_r[...],
                                   b2_r[...], w3_r[...], b3_r[...], w4_r[...])
    sh_e = jnp.dot(sh_r[...].astype(_BF), rm_r[...],
                   preferred_element_type=jnp.float32)
    up_t = jnp.dot(upg_r[...].astype(_BF), tm_r[...],
                   preferred_element_type=jnp.float32)
    msg = (w * sh_e * up_t).astype(_BF)
    oh = _onehot(eb, ow, s, e, recv_r, _BN, _CE).astype(_BF)
    contrib = jnp.dot(oh, msg, preferred_element_type=jnp.float32)

    @pl.when(init == 1)
    def _():
        out_r[...] = contrib

    @pl.when(init == 0)
    def _():
        out_r[...] += contrib


def _bwd_body(eb_r, ow_r, s_r, e_r, initf_r, initb_r,
              ef_r, sh_r, upg_r, recv_r, gagg_r,
              w1_r, b1_r, w2_r, b2_r, w3_r, b3_r, w4_r, rm_r, tm_r,
              gef_r, gsh_r, gupg_r):
    t = pl.program_id(0)
    eb = eb_r[t]
    ow = ow_r[t]
    s = s_r[t]
    e = e_r[t]
    init = initb_r[t]
    z1, h1, z2, h2, z3, h3, w = _mlp_fwd(
        ef_r[...], w1_r[...], b1_r[...], w2_r[...], b2_r[...], w3_r[...],
        b3_r[...], w4_r[...])
    rm = rm_r[...]
    tm = tm_r[...]
    sh_e = jnp.dot(sh_r[...].astype(_BF), rm,
                   preferred_element_type=jnp.float32)
    up_t = jnp.dot(upg_r[...].astype(_BF), tm,
                   preferred_element_type=jnp.float32)
    oh = _onehot(eb, ow, s, e, recv_r, _BN, _CE).astype(_BF)
    dn_c0 = (((0,), (0,)), ((), ()))
    dn_c1 = (((1,), (1,)), ((), ()))
    g_msg = jax.lax.dot_general(oh, gagg_r[...].astype(_BF), dn_c0,
                                preferred_element_type=jnp.float32)
    g_w = (g_msg * sh_e * up_t).astype(_BF)
    g_sh = jax.lax.dot_general((g_msg * w * up_t).astype(_BF), rm, dn_c1,
                               preferred_element_type=jnp.float32)
    g_upg = jax.lax.dot_general((g_msg * w * sh_e).astype(_BF), tm, dn_c1,
                                preferred_element_type=jnp.float32)
    g_h3 = jax.lax.dot_general(g_w, w4_r[...], dn_c1,
                               preferred_element_type=jnp.float32)
    g_z3 = g_h3 * _dsilu(z3)
    g_h2 = jax.lax.dot_general(g_z3.astype(_BF), w3_r[...], dn_c1,
                               preferred_element_type=jnp.float32)
    g_z2 = g_h2 * _dsilu(z2)
    g_h1 = jax.lax.dot_general(g_z2.astype(_BF), w2_r[...], dn_c1,
                               preferred_element_type=jnp.float32)
    g_z1 = g_h1 * _dsilu(z1)
    g_ef = jax.lax.dot_general(g_z1.astype(_BF), w1_r[...], dn_c1,
                               preferred_element_type=jnp.float32)

    @pl.when(init == 1)
    def _():
        gef_r[...] = g_ef
        gsh_r[...] = g_sh
        gupg_r[...] = g_upg

    @pl.when(init == 0)
    def _():
        gef_r[...] += g_ef
        gsh_r[...] += g_sh
        gupg_r[...] += g_upg


def _segsum_body(eb_r, ow_r, s_r, e_r, initf_r, initb_r,
                 vals_r, idx_r, out_r):
    t = pl.program_id(0)
    oh = _onehot(eb_r[t], ow_r[t], s_r[t], e_r[t], idx_r, _BN, _CE)
    contrib = jnp.dot(oh, vals_r[...], preferred_element_type=jnp.float32)

    @pl.when(initf_r[t] == 1)
    def _():
        out_r[...] = contrib

    @pl.when(initf_r[t] == 0)
    def _():
        out_r[...] += contrib


def _edge_spec(c):
    return pl.BlockSpec((_CE, c), lambda t, eb, ow, s, e, i_f, i_b: (eb[t], 0))


def _idx_spec():
    return pl.BlockSpec((1, 1, _CE),
                        lambda t, eb, ow, s, e, i_f, i_b: (eb[t], 0, 0))


def _win_spec():
    return pl.BlockSpec((_BN, _KF), lambda t, eb, ow, s, e, i_f, i_b: (ow[t], 0))


def _full_spec(shape):
    nd = len(shape)
    return pl.BlockSpec(shape, lambda t, eb, ow, s, e, i_f, i_b: (0,) * nd)


def _layer_weights(p, i):
    w1 = p[f"Wr1_{i}"].astype(_BF)
    w2 = p[f"Wr2_{i}"].astype(_BF)
    w3 = p[f"Wr3_{i}"].astype(_BF)
    w4 = p[f"Wr4_{i}"].astype(_BF)
    b1 = jnp.tile(p[f"br1_{i}"].reshape(1, -1), (8, 1))
    b2 = jnp.tile(p[f"br2_{i}"].reshape(1, -1), (8, 1))
    b3 = jnp.tile(p[f"br3_{i}"].reshape(1, -1), (8, 1))
    k = jnp.arange(_KF, dtype=jnp.int32)
    rm = (jnp.arange(_NUM_SH, dtype=jnp.int32)[:, None] == (k // _F)[None, :]
          ).astype(_BF)
    tm = (jnp.arange(_F, dtype=jnp.int32)[:, None] == (k % _F)[None, :]
          ).astype(_BF)
    return w1, b1, w2, b2, w3, b3, w4, rm, tm


def _fused_fwd(ef, sh, upg, recv3, wl, wts, n_nodes, l_max):
    w1, b1, w2, b2, w3, b3, w4, rm, tm = wts
    grid_spec = pltpu.PrefetchScalarGridSpec(
        num_scalar_prefetch=6,
        grid=(l_max,),
        in_specs=[
            _edge_spec(_NUM_BESSEL), _edge_spec(_NUM_SH), _edge_spec(_F),
            _idx_spec(),
            _full_spec(w1.shape), _full_spec(b1.shape),
            _full_spec(w2.shape), _full_spec(b2.shape),
            _full_spec(w3.shape), _full_spec(b3.shape),
            _full_spec(w4.shape), _full_spec(rm.shape), _full_spec(tm.shape),
        ],
        out_specs=_win_spec(),
    )
    return pl.pallas_call(
        _fwd_body, grid_spec=grid_spec,
        out_shape=jax.ShapeDtypeStruct((n_nodes, _KF), jnp.float32),
    )(*wl, ef, sh, upg, recv3, w1, b1, w2, b2, w3, b3, w4, rm, tm)


def _fused_bwd(gagg, ef, sh, upg, recv3, wl, wts, n_edges, l_max):
    w1, b1, w2, b2, w3, b3, w4, rm, tm = wts
    grid_spec = pltpu.PrefetchScalarGridSpec(
        num_scalar_prefetch=6,
        grid=(l_max,),
        in_specs=[
            _edge_spec(_NUM_BESSEL), _edge_spec(_NUM_SH), _edge_spec(_F),
            _idx_spec(), _win_spec(),
            _full_spec(w1.shape), _full_spec(b1.shape),
            _full_spec(w2.shape), _full_spec(b2.shape),
            _full_spec(w3.shape), _full_spec(b3.shape),
            _full_spec(w4.shape), _full_spec(rm.shape), _full_spec(tm.shape),
        ],
        out_specs=(_edge_spec(_NUM_BESSEL), _edge_spec(_NUM_SH),
                   _edge_spec(_F)),
    )
    return pl.pallas_call(
        _bwd_body, grid_spec=grid_spec,
        out_shape=(
            jax.ShapeDtypeStruct((n_edges, _NUM_BESSEL), jnp.float32),
            jax.ShapeDtypeStruct((n_edges, _NUM_SH), jnp.float32),
            jax.ShapeDtypeStruct((n_edges, _F), jnp.float32),
        ),
    )(*wl, ef, sh, upg, recv3, gagg, w1, b1, w2, b2, w3, b3, w4, rm, tm)


def _seg_sum(vals, idx3, wl, n_nodes, l_max):
    c = vals.shape[1]
    grid_spec = pltpu.PrefetchScalarGridSpec(
        num_scalar_prefetch=6,
        grid=(l_max,),
        in_specs=[_edge_spec(c), _idx_spec()],
        out_specs=pl.BlockSpec((_BN, c),
                               lambda t, eb, ow, s, e, i_f, i_b: (ow[t], 0)),
    )
    return pl.pallas_call(
        _segsum_body, grid_spec=grid_spec,
        out_shape=jax.ShapeDtypeStruct((n_nodes, c), jnp.float32),
    )(*wl, vals, idx3)


def kernel(node_attrs, positions, edge_index, shifts, unit_shifts, cell,
           batch, ptr, params):
    p = params
    n = positions.shape[0]
    n_edges = edge_index.shape[1]
    ngraph = cell.shape[0]
    per = n // ngraph
    nb = n_edges // _CE
    nw = n // _BN
    l_max = nb + nw

    src = edge_index[0]
    dst = edge_index[1]
    ar_e = jnp.arange(n_edges, dtype=jnp.int32)
    _, perm = jax.lax.sort_key_val(dst, ar_e)
    snd_s = src[perm]
    recv_s = dst[perm]
    shifts_s = shifts[perm]
    node_ids = jnp.arange(n + 1, dtype=jnp.int32)
    row_ptr = jnp.searchsorted(recv_s, node_ids, side="left").astype(jnp.int32)
    wl = _build_worklist(row_ptr, nw, nb, _CE, _BN)
    _, perm2 = jax.lax.sort_key_val(snd_s, ar_e)
    snd_ss = snd_s[perm2]
    row_ptr2 = jnp.searchsorted(snd_ss, node_ids, side="left").astype(jnp.int32)
    wl2 = _build_worklist(row_ptr2, nw, nb, _CE, _BN)
    recv3 = recv_s.reshape(nb, 1, _CE)
    snd3 = snd_ss.reshape(nb, 1, _CE)

    vec_s = positions[recv_s] - positions[snd_s] + shifts_s

    def edge_feats(v):
        rr = jnp.sqrt(jnp.sum(v * v, axis=-1) + 1e-9)
        return _bessel_j(rr) * _cutoff_f(rr)[:, None], _sph_f(v)

    (ef_s, sh_s), geo_vjp = jax.vjp(edge_feats, vec_s)

    feats0 = node_attrs @ p["W_embed"]
    node_e0 = node_attrs @ p["atomic_energies"]
    e0 = node_e0.reshape(ngraph, per).sum(1)

    wts0 = _layer_weights(p, 0)
    wts1 = _layer_weights(p, 1)

    up0 = feats0 @ p["Wup_0"]
    upg0 = up0[snd_s]
    agg0_raw = _fused_fwd(ef_s, sh_s, upg0, recv3, wl, wts0, n, l_max)

    w2n0 = node_attrs @ p["W2_0"]
    w3n0 = node_attrs @ p["W3_0"]
    sc0 = jnp.einsum("ne,nf,efg->ng", node_attrs, feats0, p["Wsc_0"])

    def node0(agg_raw):
        agg = agg_raw.reshape(n, _NUM_SH, _F) / _AVG_NEIGH
        aggm = jnp.einsum("nkf,fg->nkg", agg, p["Wmix_0"])
        scal = (aggm[:, 0, :] + w2n0 * jnp.sum(aggm * aggm, axis=1)
                + w3n0 * jnp.sum(aggm ** 3, axis=1))
        return scal @ p["Wprod_0"] + sc0

    feats1, pb0 = jax.vjp(node0, agg0_raw)

    up1 = feats1 @ p["Wup_1"]
    upg1 = up1[snd_s]
    agg1_raw = _fused_fwd(ef_s, sh_s, upg1, recv3, wl, wts1, n, l_max)

    w2n1 = node_attrs @ p["W2_1"]
    w3n1 = node_attrs @ p["W3_1"]

    def node1(agg_raw, f1):
        agg = agg_raw.reshape(n, _NUM_SH, _F) / _AVG_NEIGH
        aggm = jnp.einsum("nkf,fg->nkg", agg, p["Wmix_1"])
        scal = (aggm[:, 0, :] + w2n1 * jnp.sum(aggm * aggm, axis=1)
                + w3n1 * jnp.sum(aggm ** 3, axis=1))
        sc1 = jnp.einsum("ne,nf,efg->ng", node_attrs, f1, p["Wsc_1"])
        f2 = scal @ p["Wprod_1"] + sc1
        ne2v = jax.nn.silu(f2 @ p["Wm1"] + p["bm1"]) @ p["Wm2"]
        return jnp.sum(ne2v), (f2, ne2v)

    _s2, pb1, (feats2, ne2) = jax.vjp(node1, agg1_raw, feats1, has_aux=True)

    ne1 = feats1 @ p["w_ro0"]
    e1 = ne1.reshape(ngraph, per).sum(1)
    e2 = ne2.reshape(ngraph, per).sum(1)
    contributions = jnp.stack([e0, e1, e2], axis=-1)
    total = jnp.sum(contributions, axis=-1)
    node_energy = node_e0 + ne1 + ne2
    node_feats_out = jnp.concatenate([feats1, feats2], axis=-1)

    g_agg1_raw, g_feats1_a = pb1(jnp.float32(1.0))
    g_ef1, g_sh1, g_upg1 = _fused_bwd(g_agg1_raw, ef_s, sh_s, upg1, recv3, wl,
                                      wts1, n_edges, l_max)
    g_up1 = _seg_sum(g_upg1[perm2], snd3, wl2, n, l_max)
    g_feats1 = g_feats1_a + p["w_ro0"][None, :] + g_up1 @ p["Wup_1"].T
    g_agg0_raw, = pb0(g_feats1)
    g_ef0, g_sh0, _ = _fused_bwd(g_agg0_raw, ef_s, sh_s, upg0, recv3, wl,
                                 wts0, n_edges, l_max)
    g_vec, = geo_vjp((g_ef0 + g_ef1, g_sh0 + g_sh1))
    g_pos = (_seg_sum(g_vec, recv3, wl, n, l_max)
             - _seg_sum(g_vec[perm2], snd3, wl2, n, l_max))
    forces = -g_pos
    return total, node_energy, contributions, forces, node_feats_out


# speedup vs baseline: 1.0538x; 1.0538x over previous
import functools

import jax
import jax.numpy as jnp
from jax.experimental import pallas as pl
from jax.experimental.pallas import tpu as pltpu

_R_MAX = 5.0
_P_CUT = 5.0
_AVG_NEIGH = 32.0
_NUM_BESSEL = 8
_NUM_SH = 16
_F = 32
_KF = _NUM_SH * _F

_CE = 1280
_BN = 200


def _bessel_j(r):
    n = jnp.arange(1, _NUM_BESSEL + 1, dtype=jnp.float32)
    rr = jnp.clip(r, 1e-6, None)[:, None]
    return jnp.sqrt(2.0 / _R_MAX) * jnp.sin(n * jnp.pi * rr / _R_MAX) / rr


def _cutoff_f(r):
    u = r / _R_MAX
    p = _P_CUT
    f = (1.0 - (p + 1.0) * (p + 2.0) / 2.0 * u ** p
         + p * (p + 2.0) * u ** (p + 1.0)
         - p * (p + 1.0) / 2.0 * u ** (p + 2.0))
    return jnp.where(u < 1.0, f, 0.0)


def _sph_f(vec):
    r = jnp.sqrt(jnp.sum(vec * vec, axis=-1, keepdims=True) + 1e-9)
    u = vec / r
    x, y, z = u[:, 0], u[:, 1], u[:, 2]
    s3, s5, s15 = jnp.sqrt(3.0), jnp.sqrt(5.0), jnp.sqrt(15.0)
    sh = [jnp.ones_like(x),
          s3 * x, s3 * y, s3 * z,
          s15 * x * y, s15 * y * z,
          s5 / 2.0 * (3.0 * z ** 2 - 1.0),
          s15 * x * z, s15 / 2.0 * (x ** 2 - y ** 2),
          jnp.sqrt(35.0 / 8.0) * (3.0 * x ** 2 - y ** 2) * y,
          jnp.sqrt(105.0) * x * y * z,
          jnp.sqrt(21.0 / 8.0) * y * (5.0 * z ** 2 - 1.0),
          jnp.sqrt(7.0) / 2.0 * z * (5.0 * z ** 2 - 3.0),
          jnp.sqrt(21.0 / 8.0) * x * (5.0 * z ** 2 - 1.0),
          jnp.sqrt(105.0) / 2.0 * (x ** 2 - y ** 2) * z,
          jnp.sqrt(35.0 / 8.0) * (x ** 2 - 3.0 * y ** 2) * x]
    return jnp.stack(sh, axis=-1)


def _build_worklist(row_ptr, n_windows, n_blocks, ce, bn):
    w = jnp.arange(n_windows, dtype=jnp.int32)
    s_w = row_ptr[w * bn]
    e_w = row_ptr[(w + 1) * bn]
    b_start = s_w // ce
    b_end = (e_w + ce - 1) // ce
    nb = jnp.maximum(b_end - b_start, 1).astype(jnp.int32)
    off = jnp.concatenate([jnp.zeros((1,), jnp.int32),
                           jnp.cumsum(nb).astype(jnp.int32)])
    total = off[-1]
    l_max = n_blocks + n_windows
    t = jnp.arange(l_max, dtype=jnp.int32)
    wt = jnp.clip(jnp.searchsorted(off, t, side="right").astype(jnp.int32) - 1,
                  0, n_windows - 1)
    eb = jnp.clip(b_start[wt] + (t - off[wt]), 0, n_blocks - 1)
    ow = wt
    s_t = jnp.maximum(s_w[wt], eb * ce)
    e_t = jnp.minimum(e_w[wt], (eb + 1) * ce)
    valid = t < total
    s_t = jnp.where(valid, s_t, 0)
    e_t = jnp.where(valid, e_t, 0)
    ones = jnp.ones((1,), jnp.bool_)
    init_f = jnp.concatenate([ones, ow[1:] != ow[:-1]]).astype(jnp.int32)
    init_b = jnp.concatenate([ones, eb[1:] != eb[:-1]]).astype(jnp.int32)
    return eb, ow, s_t, e_t, init_f, init_b


def _dsilu(z):
    s = jax.nn.sigmoid(z)
    return s * (1.0 + z * (1.0 - s))


def _onehot(eb, ow, s, e, recv_r, bn, ce):
    recv = recv_r[0]
    base = eb * ce
    ids = base + jax.lax.broadcasted_iota(jnp.int32, (1, ce), 1)
    mask = (ids >= s) & (ids < e)
    local = recv - ow * bn
    rows = jax.lax.broadcasted_iota(jnp.int32, (bn, ce), 0)
    return ((rows == local) & mask).astype(jnp.float32)


_BF = jnp.bfloat16


def _mlp_fwd(ef, w1, b1, w2, b2, w3, b3, w4):
    z1 = jnp.dot(ef.astype(_BF), w1, preferred_element_type=jnp.float32) + b1[0:1, :]
    h1 = z1 * jax.nn.sigmoid(z1)
    z2 = jnp.dot(h1.astype(_BF), w2, preferred_element_type=jnp.float32) + b2[0:1, :]
    h2 = z2 * jax.nn.sigmoid(z2)
    z3 = jnp.dot(h2.astype(_BF), w3, preferred_element_type=jnp.float32) + b3[0:1, :]
    h3 = z3 * jax.nn.sigmoid(z3)
    w = jnp.dot(h3.astype(_BF), w4, preferred_element_type=jnp.float32)
    return z1, h1, z2, h2, z3, h3, w


def _fwd_body(eb_r, ow_r, s_r, e_r, initf_r, initb_r,
              ef_r, sh_r, upg_r, recv_r,
              w1_r, b1_r, w2_r, b2_r, w3_r, b3_r, w4_r, rm_r, tm_r,
              out_r):
    t = pl.program_id(0)
    eb = eb_r[t]
    ow = ow_r[t]
    s = s_r[t]
    e = e_r[t]
    init = initf_r[t]
    _, _, _, _, _, _, w = _mlp_fwd(ef_r[...], w1_r[...], b1_r[...], w2_r[...],
                                   b2_r[...], w3_r[...], b3_r[...], w4_r[...])
    sh_e = jnp.dot(sh_r[...].astype(_BF), rm_r[...],
                   preferred_element_type=jnp.float32)
    up_t = jnp.dot(upg_r[...].astype(_BF), tm_r[...],
                   preferred_element_type=jnp.float32)
    msg = (w * sh_e * up_t).astype(_BF)
    oh = _onehot(eb, ow, s, e, recv_r, _BN, _CE).astype(_BF)
    contrib = jnp.dot(oh, msg, preferred_element_type=jnp.float32)

    @pl.when(init == 1)
    def _():
        out_r[...] = contrib

    @pl.when(init == 0)
    def _():
        out_r[...] += contrib


def _bwd_body(eb_r, ow_r, s_r, e_r, initf_r, initb_r,
              ef_r, sh_r, upg_r, recv_r, gagg_r,
              w1_r, b1_r, w2_r, b2_r, w3_r, b3_r, w4_r, rm_r, tm_r,
              gef_r, gsh_r, gupg_r):
    t = pl.program_id(0)
    eb = eb_r[t]
    ow = ow_r[t]
    s = s_r[t]
    e = e_r[t]
    init = initb_r[t]
    z1, h1, z2, h2, z3, h3, w = _mlp_fwd(
        ef_r[...], w1_r[...], b1_r[...], w2_r[...], b2_r[...], w3_r[...],
        b3_r[...], w4_r[...])
    rm = rm_r[...]
    tm = tm_r[...]
    sh_e = jnp.dot(sh_r[...].astype(_BF), rm,
                   preferred_element_type=jnp.float32)
    up_t = jnp.dot(upg_r[...].astype(_BF), tm,
                   preferred_element_type=jnp.float32)
    oh = _onehot(eb, ow, s, e, recv_r, _BN, _CE).astype(_BF)
    dn_c0 = (((0,), (0,)), ((), ()))
    dn_c1 = (((1,), (1,)), ((), ()))
    g_msg = jax.lax.dot_general(oh, gagg_r[...].astype(_BF), dn_c0,
                                preferred_element_type=jnp.float32)
    g_w = (g_msg * sh_e * up_t).astype(_BF)
    g_sh = jax.lax.dot_general((g_msg * w * up_t).astype(_BF), rm, dn_c1,
                               preferred_element_type=jnp.float32)
    g_upg = jax.lax.dot_general((g_msg * w * sh_e).astype(_BF), tm, dn_c1,
                                preferred_element_type=jnp.float32)
    g_h3 = jax.lax.dot_general(g_w, w4_r[...], dn_c1,
                               preferred_element_type=jnp.float32)
    g_z3 = g_h3 * _dsilu(z3)
    g_h2 = jax.lax.dot_general(g_z3.astype(_BF), w3_r[...], dn_c1,
                               preferred_element_type=jnp.float32)
    g_z2 = g_h2 * _dsilu(z2)
    g_h1 = jax.lax.dot_general(g_z2.astype(_BF), w2_r[...], dn_c1,
                               preferred_element_type=jnp.float32)
    g_z1 = g_h1 * _dsilu(z1)
    g_ef = jax.lax.dot_general(g_z1.astype(_BF), w1_r[...], dn_c1,
                               preferred_element_type=jnp.float32)

    @pl.when(init == 1)
    def _():
        gef_r[...] = g_ef
        gsh_r[...] = g_sh
        gupg_r[...] = g_upg

    @pl.when(init == 0)
    def _():
        gef_r[...] += g_ef
        gsh_r[...] += g_sh
        gupg_r[...] += g_upg


def _segsum_body(eb_r, ow_r, s_r, e_r, initf_r, initb_r,
                 vals_r, idx_r, out_r):
    t = pl.program_id(0)
    oh = _onehot(eb_r[t], ow_r[t], s_r[t], e_r[t], idx_r, _BN, _CE)
    contrib = jnp.dot(oh, vals_r[...], preferred_element_type=jnp.float32)

    @pl.when(initf_r[t] == 1)
    def _():
        out_r[...] = contrib

    @pl.when(initf_r[t] == 0)
    def _():
        out_r[...] += contrib


def _edge_spec(c):
    return pl.BlockSpec((_CE, c), lambda t, eb, ow, s, e, i_f, i_b: (eb[t], 0))


def _idx_spec():
    return pl.BlockSpec((1, 1, _CE),
                        lambda t, eb, ow, s, e, i_f, i_b: (eb[t], 0, 0))


def _win_spec():
    return pl.BlockSpec((_BN, _KF), lambda t, eb, ow, s, e, i_f, i_b: (ow[t], 0))


def _full_spec(shape):
    nd = len(shape)
    return pl.BlockSpec(shape, lambda t, eb, ow, s, e, i_f, i_b: (0,) * nd)


def _layer_weights(p, i):
    w1 = p[f"Wr1_{i}"].astype(_BF)
    w2 = p[f"Wr2_{i}"].astype(_BF)
    w3 = p[f"Wr3_{i}"].astype(_BF)
    w4 = p[f"Wr4_{i}"].astype(_BF)
    b1 = jnp.tile(p[f"br1_{i}"].reshape(1, -1), (8, 1))
    b2 = jnp.tile(p[f"br2_{i}"].reshape(1, -1), (8, 1))
    b3 = jnp.tile(p[f"br3_{i}"].reshape(1, -1), (8, 1))
    k = jnp.arange(_KF, dtype=jnp.int32)
    rm = (jnp.arange(_NUM_SH, dtype=jnp.int32)[:, None] == (k // _F)[None, :]
          ).astype(_BF)
    tm = (jnp.arange(_F, dtype=jnp.int32)[:, None] == (k % _F)[None, :]
          ).astype(_BF)
    return w1, b1, w2, b2, w3, b3, w4, rm, tm


def _fused_fwd(ef, sh, upg, recv3, wl, wts, n_nodes, l_max):
    w1, b1, w2, b2, w3, b3, w4, rm, tm = wts
    grid_spec = pltpu.PrefetchScalarGridSpec(
        num_scalar_prefetch=6,
        grid=(l_max,),
        in_specs=[
            _edge_spec(_NUM_BESSEL), _edge_spec(_NUM_SH), _edge_spec(_F),
            _idx_spec(),
            _full_spec(w1.shape), _full_spec(b1.shape),
            _full_spec(w2.shape), _full_spec(b2.shape),
            _full_spec(w3.shape), _full_spec(b3.shape),
            _full_spec(w4.shape), _full_spec(rm.shape), _full_spec(tm.shape),
        ],
        out_specs=_win_spec(),
    )
    return pl.pallas_call(
        _fwd_body, grid_spec=grid_spec,
        out_shape=jax.ShapeDtypeStruct((n_nodes, _KF), jnp.float32),
    )(*wl, ef, sh, upg, recv3, w1, b1, w2, b2, w3, b3, w4, rm, tm)


def _fused_bwd(gagg, ef, sh, upg, recv3, wl, wts, n_edges, l_max):
    w1, b1, w2, b2, w3, b3, w4, rm, tm = wts
    grid_spec = pltpu.PrefetchScalarGridSpec(
        num_scalar_prefetch=6,
        grid=(l_max,),
        in_specs=[
            _edge_spec(_NUM_BESSEL), _edge_spec(_NUM_SH), _edge_spec(_F),
            _idx_spec(), _win_spec(),
            _full_spec(w1.shape), _full_spec(b1.shape),
            _full_spec(w2.shape), _full_spec(b2.shape),
            _full_spec(w3.shape), _full_spec(b3.shape),
            _full_spec(w4.shape), _full_spec(rm.shape), _full_spec(tm.shape),
        ],
        out_specs=(_edge_spec(_NUM_BESSEL), _edge_spec(_NUM_SH),
                   _edge_spec(_F)),
    )
    return pl.pallas_call(
        _bwd_body, grid_spec=grid_spec,
        out_shape=(
            jax.ShapeDtypeStruct((n_edges, _NUM_BESSEL), jnp.float32),
            jax.ShapeDtypeStruct((n_edges, _NUM_SH), jnp.float32),
            jax.ShapeDtypeStruct((n_edges, _F), jnp.float32),
        ),
    )(*wl, ef, sh, upg, recv3, gagg, w1, b1, w2, b2, w3, b3, w4, rm, tm)


def _seg_sum(vals, idx3, wl, n_nodes, l_max):
    c = vals.shape[1]
    grid_spec = pltpu.PrefetchScalarGridSpec(
        num_scalar_prefetch=6,
        grid=(l_max,),
        in_specs=[_edge_spec(c), _idx_spec()],
        out_specs=pl.BlockSpec((_BN, c),
                               lambda t, eb, ow, s, e, i_f, i_b: (ow[t], 0)),
    )
    return pl.pallas_call(
        _segsum_body, grid_spec=grid_spec,
        out_shape=jax.ShapeDtypeStruct((n_nodes, c), jnp.float32),
    )(*wl, vals, idx3)


def kernel(node_attrs, positions, edge_index, shifts, unit_shifts, cell,
           batch, ptr, params):
    p = params
    n = positions.shape[0]
    n_edges = edge_index.shape[1]
    ngraph = cell.shape[0]
    per = n // ngraph
    nb = n_edges // _CE
    nw = n // _BN
    l_max = nb + nw

    src = edge_index[0]
    dst = edge_index[1]
    ar_e = jnp.arange(n_edges, dtype=jnp.int32)
    _, perm = jax.lax.sort_key_val(dst, ar_e)
    snd_s = src[perm]
    recv_s = dst[perm]
    shifts_s = shifts[perm]
    node_ids = jnp.arange(n + 1, dtype=jnp.int32)
    row_ptr = jnp.searchsorted(recv_s, node_ids, side="left").astype(jnp.int32)
    wl = _build_worklist(row_ptr, nw, nb, _CE, _BN)
    _, perm2 = jax.lax.sort_key_val(snd_s, ar_e)
    snd_ss = snd_s[perm2]
    row_ptr2 = jnp.searchsorted(snd_ss, node_ids, side="left").astype(jnp.int32)
    wl2 = _build_worklist(row_ptr2, nw, nb, _CE, _BN)
    recv3 = recv_s.reshape(nb, 1, _CE)
    snd3 = snd_ss.reshape(nb, 1, _CE)

    vec_s = positions[recv_s] - positions[snd_s] + shifts_s

    def edge_feats(v):
        rr = jnp.sqrt(jnp.sum(v * v, axis=-1) + 1e-9)
        return _bessel_j(rr) * _cutoff_f(rr)[:, None], _sph_f(v)

    (ef_s, sh_s), geo_vjp = jax.vjp(edge_feats, vec_s)

    feats0 = node_attrs @ p["W_embed"]
    node_e0 = node_attrs @ p["atomic_energies"]
    e0 = node_e0.reshape(ngraph, per).sum(1)

    wts0 = _layer_weights(p, 0)
    wts1 = _layer_weights(p, 1)

    up0 = feats0 @ p["Wup_0"]
    upg0 = up0[snd_s]
    agg0_raw = _fused_fwd(ef_s, sh_s, upg0, recv3, wl, wts0, n, l_max)

    w2n0 = node_attrs @ p["W2_0"]
    w3n0 = node_attrs @ p["W3_0"]
    sc0 = jnp.einsum("ne,nf,efg->ng", node_attrs, feats0, p["Wsc_0"])

    def node0(agg_raw):
        agg = agg_raw.reshape(n, _NUM_SH, _F) / _AVG_NEIGH
        aggm = jnp.einsum("nkf,fg->nkg", agg, p["Wmix_0"])
        scal = (aggm[:, 0, :] + w2n0 * jnp.sum(aggm * aggm, axis=1)
                + w3n0 * jnp.sum(aggm ** 3, axis=1))
        return scal @ p["Wprod_0"] + sc0

    feats1, pb0 = jax.vjp(node0, agg0_raw)

    up1 = feats1 @ p["Wup_1"]
    upg1 = up1[snd_s]
    agg1_raw = _fused_fwd(ef_s, sh_s, upg1, recv3, wl, wts1, n, l_max)

    w2n1 = node_attrs @ p["W2_1"]
    w3n1 = node_attrs @ p["W3_1"]

    def node1(agg_raw, f1):
        agg = agg_raw.reshape(n, _NUM_SH, _F) / _AVG_NEIGH
        aggm = jnp.einsum("nkf,fg->nkg", agg, p["Wmix_1"])
        scal = (aggm[:, 0, :] + w2n1 * jnp.sum(aggm * aggm, axis=1)
                + w3n1 * jnp.sum(aggm ** 3, axis=1))
        sc1 = jnp.einsum("ne,nf,efg->ng", node_attrs, f1, p["Wsc_1"])
        f2 = scal @ p["Wprod_1"] + sc1
        ne2v = jax.nn.silu(f2 @ p["Wm1"] + p["bm1"]) @ p["Wm2"]
        return jnp.sum(ne2v), (f2, ne2v)

    _s2, pb1, (feats2, ne2) = jax.vjp(node1, agg1_raw, feats1, has_aux=True)

    ne1 = feats1 @ p["w_ro0"]
    e1 = ne1.reshape(ngraph, per).sum(1)
    e2 = ne2.reshape(ngraph, per).sum(1)
    contributions = jnp.stack([e0, e1, e2], axis=-1)
    total = jnp.sum(contributions, axis=-1)
    node_energy = node_e0 + ne1 + ne2
    node_feats_out = jnp.concatenate([feats1, feats2], axis=-1)

    g_agg1_raw, g_feats1_a = pb1(jnp.float32(1.0))
    g_ef1, g_sh1, g_upg1 = _fused_bwd(g_agg1_raw, ef_s, sh_s, upg1, recv3, wl,
                                      wts1, n_edges, l_max)
    g_up1 = _seg_sum(g_upg1[perm2], snd3, wl2, n, l_max)
    g_feats1 = g_feats1_a + p["w_ro0"][None, :] + g_up1 @ p["Wup_1"].T
    g_agg0_raw, = pb0(g_feats1)
    g_ef0, g_sh0, _ = _fused_bwd(g_agg0_raw, ef_s, sh_s, upg0, recv3, wl,
                                 wts0, n_edges, l_max)
    g_vec, = geo_vjp((g_ef0 + g_ef1, g_sh0 + g_sh1))
    g_pos = (_seg_sum(g_vec, recv3, wl, n, l_max)
             - _seg_sum(g_vec[perm2], snd3, wl2, n, l_max))
    forces = -g_pos
    return total, node_energy, contributions, forces, node_feats_out


# speedup vs baseline: 1.0587x; 1.0046x over previous
import functools

import jax
import jax.numpy as jnp
from jax.experimental import pallas as pl
from jax.experimental.pallas import tpu as pltpu

_R_MAX = 5.0
_P_CUT = 5.0
_AVG_NEIGH = 32.0
_NUM_BESSEL = 8
_NUM_SH = 16
_F = 32
_KF = _NUM_SH * _F

_CE = 1280
_BN = 200


def _bessel_j(r):
    n = jnp.arange(1, _NUM_BESSEL + 1, dtype=jnp.float32)
    rr = jnp.clip(r, 1e-6, None)[:, None]
    return jnp.sqrt(2.0 / _R_MAX) * jnp.sin(n * jnp.pi * rr / _R_MAX) / rr


def _cutoff_f(r):
    u = r / _R_MAX
    p = _P_CUT
    f = (1.0 - (p + 1.0) * (p + 2.0) / 2.0 * u ** p
         + p * (p + 2.0) * u ** (p + 1.0)
         - p * (p + 1.0) / 2.0 * u ** (p + 2.0))
    return jnp.where(u < 1.0, f, 0.0)


def _sph_f(vec):
    r = jnp.sqrt(jnp.sum(vec * vec, axis=-1, keepdims=True) + 1e-9)
    u = vec / r
    x, y, z = u[:, 0], u[:, 1], u[:, 2]
    s3, s5, s15 = jnp.sqrt(3.0), jnp.sqrt(5.0), jnp.sqrt(15.0)
    sh = [jnp.ones_like(x),
          s3 * x, s3 * y, s3 * z,
          s15 * x * y, s15 * y * z,
          s5 / 2.0 * (3.0 * z ** 2 - 1.0),
          s15 * x * z, s15 / 2.0 * (x ** 2 - y ** 2),
          jnp.sqrt(35.0 / 8.0) * (3.0 * x ** 2 - y ** 2) * y,
          jnp.sqrt(105.0) * x * y * z,
          jnp.sqrt(21.0 / 8.0) * y * (5.0 * z ** 2 - 1.0),
          jnp.sqrt(7.0) / 2.0 * z * (5.0 * z ** 2 - 3.0),
          jnp.sqrt(21.0 / 8.0) * x * (5.0 * z ** 2 - 1.0),
          jnp.sqrt(105.0) / 2.0 * (x ** 2 - y ** 2) * z,
          jnp.sqrt(35.0 / 8.0) * (x ** 2 - 3.0 * y ** 2) * x]
    return jnp.stack(sh, axis=-1)


def _build_worklist(row_ptr, n_windows, n_blocks, ce, bn):
    w = jnp.arange(n_windows, dtype=jnp.int32)
    s_w = row_ptr[w * bn]
    e_w = row_ptr[(w + 1) * bn]
    b_start = s_w // ce
    b_end = (e_w + ce - 1) // ce
    nb = jnp.maximum(b_end - b_start, 1).astype(jnp.int32)
    off = jnp.concatenate([jnp.zeros((1,), jnp.int32),
                           jnp.cumsum(nb).astype(jnp.int32)])
    total = off[-1]
    l_max = n_blocks + n_windows
    t = jnp.arange(l_max, dtype=jnp.int32)
    wt = jnp.clip(jnp.searchsorted(off, t, side="right").astype(jnp.int32) - 1,
                  0, n_windows - 1)
    eb = jnp.clip(b_start[wt] + (t - off[wt]), 0, n_blocks - 1)
    ow = wt
    s_t = jnp.maximum(s_w[wt], eb * ce)
    e_t = jnp.minimum(e_w[wt], (eb + 1) * ce)
    valid = t < total
    s_t = jnp.where(valid, s_t, 0)
    e_t = jnp.where(valid, e_t, 0)
    ones = jnp.ones((1,), jnp.bool_)
    init_f = jnp.concatenate([ones, ow[1:] != ow[:-1]]).astype(jnp.int32)
    init_b = jnp.concatenate([ones, eb[1:] != eb[:-1]]).astype(jnp.int32)
    return eb, ow, s_t, e_t, init_f, init_b


def _dsilu(z):
    s = jax.nn.sigmoid(z)
    return s * (1.0 + z * (1.0 - s))


def _onehot(eb, ow, s, e, recv_r, bn, ce):
    recv = recv_r[0]
    base = eb * ce
    ids = base + jax.lax.broadcasted_iota(jnp.int32, (1, ce), 1)
    mask = (ids >= s) & (ids < e)
    local = recv - ow * bn
    rows = jax.lax.broadcasted_iota(jnp.int32, (bn, ce), 0)
    return ((rows == local) & mask).astype(jnp.float32)


_BF = jnp.bfloat16


def _mlp_fwd(ef, w1, b1, w2, b2, w3, b3, w4):
    z1 = jnp.dot(ef.astype(_BF), w1, preferred_element_type=jnp.float32) + b1[0:1, :]
    h1 = z1 * jax.nn.sigmoid(z1)
    z2 = jnp.dot(h1.astype(_BF), w2, preferred_element_type=jnp.float32) + b2[0:1, :]
    h2 = z2 * jax.nn.sigmoid(z2)
    z3 = jnp.dot(h2.astype(_BF), w3, preferred_element_type=jnp.float32) + b3[0:1, :]
    h3 = z3 * jax.nn.sigmoid(z3)
    w = jnp.dot(h3.astype(_BF), w4, preferred_element_type=jnp.float32)
    return z1, h1, z2, h2, z3, h3, w


def _fwd_body(eb_r, ow_r, s_r, e_r, initf_r, initb_r,
              ef_r, sh_r, upg_r, recv_r,
              w1_r, b1_r, w2_r, b2_r, w3_r, b3_r, w4_r, rm_r, tm_r,
              out_r):
    t = pl.program_id(0)
    eb = eb_r[t]
    ow = ow_r[t]
    s = s_r[t]
    e = e_r[t]
    init = initf_r[t]
    _, _, _, _, _, _, w = _mlp_fwd(ef_r[...], w1_r[...], b1_r[...], w2_r[...],
                                   b2_r[...], w3_r[...], b3_r[...], w4_r[...])
    sh_e = jnp.dot(sh_r[...].astype(_BF), rm_r[...],
                   preferred_element_type=jnp.float32)
    up_t = jnp.dot(upg_r[...].astype(_BF), tm_r[...],
                   preferred_element_type=jnp.float32)
    msg = (w * sh_e * up_t).astype(_BF)
    oh = _onehot(eb, ow, s, e, recv_r, _BN, _CE).astype(_BF)
    contrib = jnp.dot(oh, msg, preferred_element_type=jnp.float32)

    @pl.when(init == 1)
    def _():
        out_r[...] = contrib

    @pl.when(init == 0)
    def _():
        out_r[...] += contrib


def _bwd_body(eb_r, ow_r, s_r, e_r, initf_r, initb_r,
              ef_r, sh_r, upg_r, recv_r, gagg_r,
              w1_r, b1_r, w2_r, b2_r, w3_r, b3_r, w4_r, rm_r, tm_r,
              gef_r, gsh_r, gupg_r):
    t = pl.program_id(0)
    eb = eb_r[t]
    ow = ow_r[t]
    s = s_r[t]
    e = e_r[t]
    init = initb_r[t]
    z1, h1, z2, h2, z3, h3, w = _mlp_fwd(
        ef_r[...], w1_r[...], b1_r[...], w2_r[...], b2_r[...], w3_r[...],
        b3_r[...], w4_r[...])
    rm = rm_r[...]
    tm = tm_r[...]
    sh_e = jnp.dot(sh_r[...].astype(_BF), rm,
                   preferred_element_type=jnp.float32)
    up_t = jnp.dot(upg_r[...].astype(_BF), tm,
                   preferred_element_type=jnp.float32)
    oh = _onehot(eb, ow, s, e, recv_r, _BN, _CE).astype(_BF)
    dn_c0 = (((0,), (0,)), ((), ()))
    dn_c1 = (((1,), (1,)), ((), ()))
    g_msg = jax.lax.dot_general(oh, gagg_r[...].astype(_BF), dn_c0,
                                preferred_element_type=jnp.float32)
    g_w = (g_msg * sh_e * up_t).astype(_BF)
    g_sh = jax.lax.dot_general((g_msg * w * up_t).astype(_BF), rm, dn_c1,
                               preferred_element_type=jnp.float32)
    g_upg = jax.lax.dot_general((g_msg * w * sh_e).astype(_BF), tm, dn_c1,
                                preferred_element_type=jnp.float32)
    g_h3 = jax.lax.dot_general(g_w, w4_r[...], dn_c1,
                               preferred_element_type=jnp.float32)
    g_z3 = g_h3 * _dsilu(z3)
    g_h2 = jax.lax.dot_general(g_z3.astype(_BF), w3_r[...], dn_c1,
                               preferred_element_type=jnp.float32)
    g_z2 = g_h2 * _dsilu(z2)
    g_h1 = jax.lax.dot_general(g_z2.astype(_BF), w2_r[...], dn_c1,
                               preferred_element_type=jnp.float32)
    g_z1 = g_h1 * _dsilu(z1)
    g_ef = jax.lax.dot_general(g_z1.astype(_BF), w1_r[...], dn_c1,
                               preferred_element_type=jnp.float32)

    @pl.when(init == 1)
    def _():
        gef_r[...] = g_ef
        gsh_r[...] = g_sh
        gupg_r[...] = g_upg

    @pl.when(init == 0)
    def _():
        gef_r[...] += g_ef
        gsh_r[...] += g_sh
        gupg_r[...] += g_upg


def _segsum_body(eb_r, ow_r, s_r, e_r, initf_r, initb_r,
                 vals_r, idx_r, out_r):
    t = pl.program_id(0)
    oh = _onehot(eb_r[t], ow_r[t], s_r[t], e_r[t], idx_r, _BN, _CE)
    contrib = jnp.dot(oh, vals_r[...], preferred_element_type=jnp.float32)

    @pl.when(initf_r[t] == 1)
    def _():
        out_r[...] = contrib

    @pl.when(initf_r[t] == 0)
    def _():
        out_r[...] += contrib


def _segsum2_body(eb_r, ow_r, s_r, e_r, initf_r, initb_r,
                  eb2_r, ow2_r, s2_r, e2_r, initf2_r, initb2_r,
                  va_r, ia_r, vb_r, ib_r, outa_r, outb_r):
    t = pl.program_id(0)
    oha = _onehot(eb_r[t], ow_r[t], s_r[t], e_r[t], ia_r, _BN, _CE)
    ca = jnp.dot(oha, va_r[...], preferred_element_type=jnp.float32)
    ohb = _onehot(eb2_r[t], ow2_r[t], s2_r[t], e2_r[t], ib_r, _BN, _CE)
    cb = jnp.dot(ohb, vb_r[...], preferred_element_type=jnp.float32)

    @pl.when(initf_r[t] == 1)
    def _():
        outa_r[...] = ca

    @pl.when(initf_r[t] == 0)
    def _():
        outa_r[...] += ca

    @pl.when(initf2_r[t] == 1)
    def _():
        outb_r[...] = cb

    @pl.when(initf2_r[t] == 0)
    def _():
        outb_r[...] += cb


def _seg_sum2(vals_a, idx3_a, vals_b, idx3_b, wl, wl2, n_nodes, l_max):
    c = vals_a.shape[1]
    p12 = 12 * (None,)

    def _es(j):
        return pl.BlockSpec((_CE, c), lambda t, *pf, _j=j: (pf[_j][t], 0))

    def _is(j):
        return pl.BlockSpec((1, 1, _CE), lambda t, *pf, _j=j: (pf[_j][t], 0, 0))

    def _os(j):
        return pl.BlockSpec((_BN, c), lambda t, *pf, _j=j: (pf[_j][t], 0))

    grid_spec = pltpu.PrefetchScalarGridSpec(
        num_scalar_prefetch=12,
        grid=(l_max,),
        in_specs=[_es(0), _is(0), _es(6), _is(6)],
        out_specs=(_os(1), _os(7)),
    )
    return pl.pallas_call(
        _segsum2_body, grid_spec=grid_spec,
        out_shape=(jax.ShapeDtypeStruct((n_nodes, c), jnp.float32),
                   jax.ShapeDtypeStruct((n_nodes, c), jnp.float32)),
    )(*wl, *wl2, vals_a, idx3_a, vals_b, idx3_b)


def _edge_spec(c):
    return pl.BlockSpec((_CE, c), lambda t, eb, ow, s, e, i_f, i_b: (eb[t], 0))


def _idx_spec():
    return pl.BlockSpec((1, 1, _CE),
                        lambda t, eb, ow, s, e, i_f, i_b: (eb[t], 0, 0))


def _win_spec():
    return pl.BlockSpec((_BN, _KF), lambda t, eb, ow, s, e, i_f, i_b: (ow[t], 0))


def _full_spec(shape):
    nd = len(shape)
    return pl.BlockSpec(shape, lambda t, eb, ow, s, e, i_f, i_b: (0,) * nd)


def _layer_weights(p, i):
    w1 = p[f"Wr1_{i}"].astype(_BF)
    w2 = p[f"Wr2_{i}"].astype(_BF)
    w3 = p[f"Wr3_{i}"].astype(_BF)
    w4 = p[f"Wr4_{i}"].astype(_BF)
    b1 = jnp.tile(p[f"br1_{i}"].reshape(1, -1), (8, 1))
    b2 = jnp.tile(p[f"br2_{i}"].reshape(1, -1), (8, 1))
    b3 = jnp.tile(p[f"br3_{i}"].reshape(1, -1), (8, 1))
    k = jnp.arange(_KF, dtype=jnp.int32)
    rm = (jnp.arange(_NUM_SH, dtype=jnp.int32)[:, None] == (k // _F)[None, :]
          ).astype(_BF)
    tm = (jnp.arange(_F, dtype=jnp.int32)[:, None] == (k % _F)[None, :]
          ).astype(_BF)
    return w1, b1, w2, b2, w3, b3, w4, rm, tm


def _fused_fwd(ef, sh, upg, recv3, wl, wts, n_nodes, l_max):
    w1, b1, w2, b2, w3, b3, w4, rm, tm = wts
    grid_spec = pltpu.PrefetchScalarGridSpec(
        num_scalar_prefetch=6,
        grid=(l_max,),
        in_specs=[
            _edge_spec(_NUM_BESSEL), _edge_spec(_NUM_SH), _edge_spec(_F),
            _idx_spec(),
            _full_spec(w1.shape), _full_spec(b1.shape),
            _full_spec(w2.shape), _full_spec(b2.shape),
            _full_spec(w3.shape), _full_spec(b3.shape),
            _full_spec(w4.shape), _full_spec(rm.shape), _full_spec(tm.shape),
        ],
        out_specs=_win_spec(),
    )
    return pl.pallas_call(
        _fwd_body, grid_spec=grid_spec,
        out_shape=jax.ShapeDtypeStruct((n_nodes, _KF), jnp.float32),
    )(*wl, ef, sh, upg, recv3, w1, b1, w2, b2, w3, b3, w4, rm, tm)


def _fused_bwd(gagg, ef, sh, upg, recv3, wl, wts, n_edges, l_max):
    w1, b1, w2, b2, w3, b3, w4, rm, tm = wts
    grid_spec = pltpu.PrefetchScalarGridSpec(
        num_scalar_prefetch=6,
        grid=(l_max,),
        in_specs=[
            _edge_spec(_NUM_BESSEL), _edge_spec(_NUM_SH), _edge_spec(_F),
            _idx_spec(), _win_spec(),
            _full_spec(w1.shape), _full_spec(b1.shape),
            _full_spec(w2.shape), _full_spec(b2.shape),
            _full_spec(w3.shape), _full_spec(b3.shape),
            _full_spec(w4.shape), _full_spec(rm.shape), _full_spec(tm.shape),
        ],
        out_specs=(_edge_spec(_NUM_BESSEL), _edge_spec(_NUM_SH),
                   _edge_spec(_F)),
    )
    return pl.pallas_call(
        _bwd_body, grid_spec=grid_spec,
        out_shape=(
            jax.ShapeDtypeStruct((n_edges, _NUM_BESSEL), jnp.float32),
            jax.ShapeDtypeStruct((n_edges, _NUM_SH), jnp.float32),
            jax.ShapeDtypeStruct((n_edges, _F), jnp.float32),
        ),
    )(*wl, ef, sh, upg, recv3, gagg, w1, b1, w2, b2, w3, b3, w4, rm, tm)


def _seg_sum(vals, idx3, wl, n_nodes, l_max):
    c = vals.shape[1]
    grid_spec = pltpu.PrefetchScalarGridSpec(
        num_scalar_prefetch=6,
        grid=(l_max,),
        in_specs=[_edge_spec(c), _idx_spec()],
        out_specs=pl.BlockSpec((_BN, c),
                               lambda t, eb, ow, s, e, i_f, i_b: (ow[t], 0)),
    )
    return pl.pallas_call(
        _segsum_body, grid_spec=grid_spec,
        out_shape=jax.ShapeDtypeStruct((n_nodes, c), jnp.float32),
    )(*wl, vals, idx3)


def kernel(node_attrs, positions, edge_index, shifts, unit_shifts, cell,
           batch, ptr, params):
    p = params
    n = positions.shape[0]
    n_edges = edge_index.shape[1]
    ngraph = cell.shape[0]
    per = n // ngraph
    nb = n_edges // _CE
    nw = n // _BN
    l_max = nb + nw

    src = edge_index[0]
    dst = edge_index[1]
    ar_e = jnp.arange(n_edges, dtype=jnp.int32)
    _, perm = jax.lax.sort_key_val(dst, ar_e)
    snd_s = src[perm]
    recv_s = dst[perm]
    shifts_s = shifts[perm]
    node_ids = jnp.arange(n + 1, dtype=jnp.int32)
    row_ptr = jnp.searchsorted(recv_s, node_ids, side="left").astype(jnp.int32)
    wl = _build_worklist(row_ptr, nw, nb, _CE, _BN)
    _, perm2 = jax.lax.sort_key_val(snd_s, ar_e)
    snd_ss = snd_s[perm2]
    row_ptr2 = jnp.searchsorted(snd_ss, node_ids, side="left").astype(jnp.int32)
    wl2 = _build_worklist(row_ptr2, nw, nb, _CE, _BN)
    recv3 = recv_s.reshape(nb, 1, _CE)
    snd3 = snd_ss.reshape(nb, 1, _CE)

    vec_s = positions[recv_s] - positions[snd_s] + shifts_s

    def edge_feats(v):
        rr = jnp.sqrt(jnp.sum(v * v, axis=-1) + 1e-9)
        return _bessel_j(rr) * _cutoff_f(rr)[:, None], _sph_f(v)

    (ef_s, sh_s), geo_vjp = jax.vjp(edge_feats, vec_s)

    feats0 = node_attrs @ p["W_embed"]
    node_e0 = node_attrs @ p["atomic_energies"]
    e0 = node_e0.reshape(ngraph, per).sum(1)

    wts0 = _layer_weights(p, 0)
    wts1 = _layer_weights(p, 1)

    up0 = feats0 @ p["Wup_0"]
    upg0 = up0[snd_s]
    agg0_raw = _fused_fwd(ef_s, sh_s, upg0, recv3, wl, wts0, n, l_max)

    w2n0 = node_attrs @ p["W2_0"]
    w3n0 = node_attrs @ p["W3_0"]
    sc0 = jnp.einsum("ne,nf,efg->ng", node_attrs, feats0, p["Wsc_0"])

    def node0(agg_raw):
        agg = agg_raw.reshape(n, _NUM_SH, _F) / _AVG_NEIGH
        aggm = jnp.einsum("nkf,fg->nkg", agg, p["Wmix_0"])
        scal = (aggm[:, 0, :] + w2n0 * jnp.sum(aggm * aggm, axis=1)
                + w3n0 * jnp.sum(aggm ** 3, axis=1))
        return scal @ p["Wprod_0"] + sc0

    feats1, pb0 = jax.vjp(node0, agg0_raw)

    up1 = feats1 @ p["Wup_1"]
    upg1 = up1[snd_s]
    agg1_raw = _fused_fwd(ef_s, sh_s, upg1, recv3, wl, wts1, n, l_max)

    w2n1 = node_attrs @ p["W2_1"]
    w3n1 = node_attrs @ p["W3_1"]

    def node1(agg_raw, f1):
        agg = agg_raw.reshape(n, _NUM_SH, _F) / _AVG_NEIGH
        aggm = jnp.einsum("nkf,fg->nkg", agg, p["Wmix_1"])
        scal = (aggm[:, 0, :] + w2n1 * jnp.sum(aggm * aggm, axis=1)
                + w3n1 * jnp.sum(aggm ** 3, axis=1))
        sc1 = jnp.einsum("ne,nf,efg->ng", node_attrs, f1, p["Wsc_1"])
        f2 = scal @ p["Wprod_1"] + sc1
        ne2v = jax.nn.silu(f2 @ p["Wm1"] + p["bm1"]) @ p["Wm2"]
        return jnp.sum(ne2v), (f2, ne2v)

    _s2, pb1, (feats2, ne2) = jax.vjp(node1, agg1_raw, feats1, has_aux=True)

    ne1 = feats1 @ p["w_ro0"]
    e1 = ne1.reshape(ngraph, per).sum(1)
    e2 = ne2.reshape(ngraph, per).sum(1)
    contributions = jnp.stack([e0, e1, e2], axis=-1)
    total = jnp.sum(contributions, axis=-1)
    node_energy = node_e0 + ne1 + ne2
    node_feats_out = jnp.concatenate([feats1, feats2], axis=-1)

    g_agg1_raw, g_feats1_a = pb1(jnp.float32(1.0))
    g_ef1, g_sh1, g_upg1 = _fused_bwd(g_agg1_raw, ef_s, sh_s, upg1, recv3, wl,
                                      wts1, n_edges, l_max)
    g_up1 = _seg_sum(g_upg1[perm2], snd3, wl2, n, l_max)
    g_feats1 = g_feats1_a + p["w_ro0"][None, :] + g_up1 @ p["Wup_1"].T
    g_agg0_raw, = pb0(g_feats1)
    g_ef0, g_sh0, _ = _fused_bwd(g_agg0_raw, ef_s, sh_s, upg0, recv3, wl,
                                 wts0, n_edges, l_max)
    g_vec, = geo_vjp((g_ef0 + g_ef1, g_sh0 + g_sh1))
    gp_recv, gp_snd = _seg_sum2(g_vec, recv3, g_vec[perm2], snd3,
                                wl, wl2, n, l_max)
    g_pos = gp_recv - gp_snd
    forces = -g_pos
    return total, node_energy, contributions, forces, node_feats_out
